# Initial kernel scaffold; baseline (speedup 1.0000x reference)
import jax
import jax.numpy as jnp
from jax.experimental import pallas as pl


def kernel(pos, hidden, W, b):
    raise NotImplementedError("write your pallas kernel here")


# speedup vs baseline: 10.7699x; 10.7699x over previous
import jax
import jax.numpy as jnp
from jax import lax
from jax.experimental import pallas as pl
from jax.experimental.pallas import tpu as pltpu
from jax.experimental.pallas import tpu_sc as plsc

_N = 512
_H = 128
_P = 128
_G = 4
_GG = _G * _G
_NW = 32
_APW = _N // _NW
_L = 16
_NC = _N // _L


def _pool_body(posx_hbm, posy_hbm, posx1_hbm, posy1_hbm, hid_hbm, out_hbm,
               posx_v, posy_v, posx1_v, posy1_v, hid_v, sl_v, acc_v):
    cid = lax.axis_index("c")
    sid = lax.axis_index("s")
    wid = sid * 2 + cid

    pltpu.sync_copy(posx_hbm, posx_v)
    pltpu.sync_copy(posy_hbm, posy_v)
    pltpu.sync_copy(posx1_hbm, posx1_v)
    pltpu.sync_copy(posy1_hbm, posy1_v)
    pltpu.sync_copy(hid_hbm, hid_v)

    lanes = lax.iota(jnp.int32, _L)

    def per_agent(a, _):
        i = wid * _APW + a
        iv = jnp.full((_L,), i, jnp.int32)
        pxi = plsc.load_gather(posx1_v, [iv])
        pyi = plsc.load_gather(posy1_v, [iv])

        def chunk(jc, m):
            px = posx_v[jc, :]
            py = posy_v[jc, :]
            relx = px - pxi
            rely = py - pyi
            inb = (jnp.abs(relx) <= 1.0) & (jnp.abs(rely) <= 1.0)
            jv = jc * _L + lanes
            valid = inb & (jv != i)
            gx = jnp.clip((relx + 1.0) * 2.0, 0.0, _G - 1.0).astype(jnp.int32)
            gy = jnp.clip((rely + 1.0) * 2.0, 0.0, _G - 1.0).astype(jnp.int32)
            val = (gx * _G + gy) * _N + jv
            vi = valid.astype(jnp.int32)
            offs = m + plsc.cumsum(vi) - 1
            plsc.store_scatter(sl_v, [offs], val, mask=valid)
            return m + jnp.sum(vi)

        nv = lax.fori_loop(0, _NC, chunk, jnp.int32(0))

        def initr(r, _):
            for hc in range(_H // _L):
                acc_v[r, pl.ds(hc * _L, _L)] = jnp.full((_L,), -3e38,
                                                        jnp.float32)
            return 0

        lax.fori_loop(0, _GG, initr, 0)

        def upd(k, _):
            valv = plsc.load_gather(sl_v, [jnp.full((_L,), k, jnp.int32)])
            vs = jnp.max(valv)
            c = vs >> 9
            j = vs & (_N - 1)
            for hc in range(_H // _L):
                s = pl.ds(hc * _L, _L)
                acc_v[c, s] = jnp.maximum(acc_v[c, s], hid_v[j, s])
            return 0

        lax.fori_loop(0, nv, upd, 0)

        def fin(r, _):
            for hc in range(_H // _L):
                s = pl.ds(hc * _L, _L)
                v = acc_v[r, s]
                acc_v[r, s] = jnp.where(v < -1e37, 0.0, v)
            return 0

        lax.fori_loop(0, _GG, fin, 0)
        pltpu.sync_copy(acc_v, out_hbm.at[i])
        return 0

    lax.fori_loop(0, _APW, per_agent, 0)


def _mm_body(g_ref, w_ref, b_ref, o_ref):
    o_ref[...] = jnp.maximum(
        jnp.dot(g_ref[...], w_ref[...], preferred_element_type=jnp.float32)
        + b_ref[...],
        0.0,
    )


def kernel(pos, hidden, W, b):
    posx = pos[:, 0].reshape(_NC, _L)
    posy = pos[:, 1].reshape(_NC, _L)

    grid = pl.kernel(
        _pool_body,
        out_type=jax.ShapeDtypeStruct((_N, _GG, _H), jnp.float32),
        mesh=plsc.VectorSubcoreMesh(core_axis_name="c", subcore_axis_name="s"),
        scratch_types=[
            pltpu.VMEM((_NC, _L), jnp.float32),
            pltpu.VMEM((_NC, _L), jnp.float32),
            pltpu.VMEM((_N,), jnp.float32),
            pltpu.VMEM((_N,), jnp.float32),
            pltpu.VMEM((_N, _H), jnp.float32),
            pltpu.VMEM((_N,), jnp.int32),
            pltpu.VMEM((_GG, _H), jnp.float32),
        ],
        compiler_params=pltpu.CompilerParams(needs_layout_passes=False),
    )(posx, posy, posx.reshape(_N), posy.reshape(_N), hidden)

    return pl.pallas_call(
        _mm_body,
        out_shape=jax.ShapeDtypeStruct((_N, _P), jnp.float32),
    )(grid.reshape(_N, _GG * _H), W, b.reshape(1, _P))


# speedup vs baseline: 13.8515x; 1.2861x over previous
import jax
import jax.numpy as jnp
from jax import lax
from jax.experimental import pallas as pl
from jax.experimental.pallas import tpu as pltpu
from jax.experimental.pallas import tpu_sc as plsc

_N = 512
_H = 128
_P = 128
_G = 4
_GG = _G * _G
_NW = 32
_APW = _N // _NW
_L = 16
_NC = _N // _L


def _pool_body(posx_hbm, posy_hbm, posx1_hbm, posy1_hbm, hid_hbm, out_hbm,
               posx_v, posy_v, posx1_v, posy1_v, hid_v, sl_v, acc_v):
    cid = lax.axis_index("c")
    sid = lax.axis_index("s")
    wid = sid * 2 + cid

    pltpu.sync_copy(posx_hbm, posx_v)
    pltpu.sync_copy(posy_hbm, posy_v)
    pltpu.sync_copy(posx1_hbm, posx1_v)
    pltpu.sync_copy(posy1_hbm, posy1_v)
    pltpu.sync_copy(hid_hbm, hid_v)

    lanes = lax.iota(jnp.int32, _L)

    def per_agent(a, _):
        i = wid * _APW + a
        iv = jnp.full((_L,), i, jnp.int32)
        pxi = plsc.load_gather(posx1_v, [iv])
        pyi = plsc.load_gather(posy1_v, [iv])

        def chunk(jc, m):
            px = posx_v[jc, :]
            py = posy_v[jc, :]
            relx = px - pxi
            rely = py - pyi
            inb = (jnp.abs(relx) <= 1.0) & (jnp.abs(rely) <= 1.0)
            jv = jc * _L + lanes
            valid = inb & (jv != i)
            gx = jnp.clip((relx + 1.0) * 2.0, 0.0, _G - 1.0).astype(jnp.int32)
            gy = jnp.clip((rely + 1.0) * 2.0, 0.0, _G - 1.0).astype(jnp.int32)
            val = (gx * _G + gy) * _N + jv
            vi = valid.astype(jnp.int32)
            offs = m + plsc.cumsum(vi) - 1
            plsc.store_scatter(sl_v, [offs], val, mask=valid)
            return m + jnp.sum(vi)

        nv = lax.fori_loop(0, _NC, chunk, jnp.int32(0))

        plsc.store_scatter(sl_v, [nv + lanes],
                           jnp.full((_L,), _GG * _N, jnp.int32))

        def initr(r, _):
            for hc in range(_H // _L):
                acc_v[r, pl.ds(hc * _L, _L)] = jnp.full((_L,), -3e38,
                                                        jnp.float32)
            return 0

        lax.fori_loop(0, _GG, initr, 0)

        def upd(kc, _):
            valv = sl_v[pl.ds(kc * _L, _L)]
            for l in range(_L):
                vs = valv[l]
                c = vs >> 9
                j = vs & (_N - 1)
                for hc in range(_H // _L):
                    s = pl.ds(hc * _L, _L)
                    acc_v[c, s] = jnp.maximum(acc_v[c, s], hid_v[j, s])
            return 0

        lax.fori_loop(0, (nv + _L - 1) >> 4, upd, 0)

        def fin(r, _):
            for hc in range(_H // _L):
                s = pl.ds(hc * _L, _L)
                v = acc_v[r, s]
                acc_v[r, s] = jnp.where(v < -1e37, 0.0, v)
            return 0

        lax.fori_loop(0, _GG, fin, 0)
        pltpu.sync_copy(acc_v.at[pl.ds(0, _GG)], out_hbm.at[i])
        return 0

    lax.fori_loop(0, _APW, per_agent, 0)


def _mm_body(g_ref, w_ref, b_ref, o_ref):
    o_ref[...] = jnp.maximum(
        jnp.dot(g_ref[...], w_ref[...], preferred_element_type=jnp.float32)
        + b_ref[...],
        0.0,
    )


def kernel(pos, hidden, W, b):
    posx = pos[:, 0].reshape(_NC, _L)
    posy = pos[:, 1].reshape(_NC, _L)

    grid = pl.kernel(
        _pool_body,
        out_type=jax.ShapeDtypeStruct((_N, _GG, _H), jnp.float32),
        mesh=plsc.VectorSubcoreMesh(core_axis_name="c", subcore_axis_name="s"),
        scratch_types=[
            pltpu.VMEM((_NC, _L), jnp.float32),
            pltpu.VMEM((_NC, _L), jnp.float32),
            pltpu.VMEM((_N,), jnp.float32),
            pltpu.VMEM((_N,), jnp.float32),
            pltpu.VMEM((_N, _H), jnp.float32),
            pltpu.VMEM((_N + _L,), jnp.int32),
            pltpu.VMEM((_GG + 1, _H), jnp.float32),
        ],
        compiler_params=pltpu.CompilerParams(needs_layout_passes=False),
    )(posx, posy, posx.reshape(_N), posy.reshape(_N), hidden)

    return pl.pallas_call(
        _mm_body,
        out_shape=jax.ShapeDtypeStruct((_N, _P), jnp.float32),
    )(grid.reshape(_N, _GG * _H), W, b.reshape(1, _P))
